# baseline (speedup 1.0000x reference)
import functools

import jax
import jax.numpy as jnp
from jax import lax
from jax.experimental import pallas as pl
from jax.experimental.pallas import tpu as pltpu
from jax.experimental.pallas import tpu_sc as plsc

VOCAB = 1000000
D_MODEL = 64
BATCH = 4096
SEQ = 200

_INFO = plsc.get_sparse_core_info()
_NC = _INFO.num_cores
_NS = _INFO.num_subcores
_NW = _NC * _NS
_B = BATCH * SEQ
_B_PER_W = _B // _NW
_CHUNK = 512
_N_CHUNKS = _B_PER_W // _CHUNK

assert _B % _NW == 0 and _B_PER_W % _CHUNK == 0


def _gather_sc(table, idx_flat):
    mesh = plsc.VectorSubcoreMesh(core_axis_name="c", subcore_axis_name="s")

    @functools.partial(
        pl.kernel,
        mesh=mesh,
        out_type=jax.ShapeDtypeStruct((_B, D_MODEL), jnp.float32),
        scratch_types=[
            pltpu.VMEM((_CHUNK,), jnp.int32),
            pltpu.VMEM((_CHUNK, D_MODEL), jnp.float32),
            pltpu.SemaphoreType.DMA,
        ],
        compiler_params=pltpu.CompilerParams(use_tc_tiling_on_sc=False),
    )
    def k(table_hbm, idx_hbm, out_hbm, idx_v, rows_v, sem):
        wid = lax.axis_index("s") * _NC + lax.axis_index("c")
        base = wid * _B_PER_W

        def body(i, carry):
            off = base + i * _CHUNK
            pltpu.sync_copy(idx_hbm.at[pl.ds(off, _CHUNK)], idx_v)
            pltpu.async_copy(table_hbm.at[idx_v], rows_v, sem).wait()
            pltpu.sync_copy(rows_v, out_hbm.at[pl.ds(off, _CHUNK)])
            return carry

        lax.fori_loop(0, _N_CHUNKS, body, 0)

    return k(table, idx_flat)


def kernel(input_ids, shared_weight):
    idx_flat = input_ids.reshape(-1).astype(jnp.int32)
    out = _gather_sc(shared_weight, idx_flat)
    return out.reshape(*input_ids.shape, D_MODEL)


# speedup vs baseline: 1.0453x; 1.0453x over previous
import functools

import jax
import jax.numpy as jnp
from jax import lax
from jax.experimental import pallas as pl
from jax.experimental.pallas import tpu as pltpu
from jax.experimental.pallas import tpu_sc as plsc

VOCAB = 1000000
D_MODEL = 64
BATCH = 4096
SEQ = 200

_INFO = plsc.get_sparse_core_info()
_NC = _INFO.num_cores
_NS = _INFO.num_subcores
_NW = _NC * _NS
_B = BATCH * SEQ
_B_PER_W = _B // _NW
_CHUNK = 256
_N_CHUNKS = _B_PER_W // _CHUNK
_NBUF = 4
_LOOKAHEAD = _NBUF // 2

assert _B % _NW == 0 and _B_PER_W % _CHUNK == 0 and _N_CHUNKS > _NBUF


def _gather_sc(table, idx_flat):
    mesh = plsc.VectorSubcoreMesh(core_axis_name="c", subcore_axis_name="s")

    @functools.partial(
        pl.kernel,
        mesh=mesh,
        out_type=jax.ShapeDtypeStruct((_B, D_MODEL), jnp.float32),
        scratch_types=[
            pltpu.VMEM((_B_PER_W,), jnp.int32),
        ] + [pltpu.VMEM((_CHUNK, D_MODEL), jnp.float32) for _ in range(_NBUF)]
          + [pltpu.SemaphoreType.DMA for _ in range(2 * _NBUF)],
        compiler_params=pltpu.CompilerParams(use_tc_tiling_on_sc=False),
    )
    def k(table_hbm, idx_hbm, out_hbm, idx_all, *bufs_and_sems):
        bufs = bufs_and_sems[:_NBUF]
        gsems = bufs_and_sems[_NBUF:2 * _NBUF]
        wsems = bufs_and_sems[2 * _NBUF:]
        wid = lax.axis_index("s") * _NC + lax.axis_index("c")
        base = wid * _B_PER_W

        pltpu.sync_copy(idx_hbm.at[pl.ds(base, _B_PER_W)], idx_all)

        def start_gather(chunk, b):
            pltpu.make_async_copy(
                table_hbm.at[idx_all.at[pl.ds(chunk * _CHUNK, _CHUNK)]],
                bufs[b], gsems[b]).start()

        def wait_gather(chunk, b):
            pltpu.make_async_copy(
                table_hbm.at[idx_all.at[pl.ds(chunk * _CHUNK, _CHUNK)]],
                bufs[b], gsems[b]).wait()

        def start_write(chunk, b):
            pltpu.make_async_copy(
                bufs[b], out_hbm.at[pl.ds(base + chunk * _CHUNK, _CHUNK)],
                wsems[b]).start()

        def wait_write(chunk, b):
            pltpu.make_async_copy(
                bufs[b], out_hbm.at[pl.ds(base + chunk * _CHUNK, _CHUNK)],
                wsems[b]).wait()

        for j in range(_LOOKAHEAD):
            start_gather(j, j % _NBUF)

        def body(g, carry):
            for bb in range(_NBUF):
                j = g * _NBUF + bb
                wait_gather(j, bb)
                start_write(j, bb)
                t = j + _LOOKAHEAD
                bt = (bb + _LOOKAHEAD) % _NBUF

                @pl.when(t >= _NBUF)
                def _(t=t, bt=bt):
                    wait_write(t - _NBUF, bt)

                @pl.when(t < _N_CHUNKS)
                def _(t=t, bt=bt):
                    start_gather(t, bt)
            return carry

        lax.fori_loop(0, _N_CHUNKS // _NBUF, body, 0)

        for j in range(_N_CHUNKS - _LOOKAHEAD, _N_CHUNKS):
            wait_write(j, j % _NBUF)

    return k(table, idx_flat)


def kernel(input_ids, shared_weight):
    idx_flat = input_ids.reshape(-1).astype(jnp.int32)
    out = _gather_sc(shared_weight, idx_flat)
    return out.reshape(*input_ids.shape, D_MODEL)


# speedup vs baseline: 1.2729x; 1.2177x over previous
import functools

import jax
import jax.numpy as jnp
from jax import lax
from jax.experimental import pallas as pl
from jax.experimental.pallas import tpu as pltpu
from jax.experimental.pallas import tpu_sc as plsc

VOCAB = 1000000
D_MODEL = 64
BATCH = 4096
SEQ = 200

_INFO = plsc.get_sparse_core_info()
_NC = _INFO.num_cores
_NS = _INFO.num_subcores
_NW = _NC * _NS
_B = BATCH * SEQ
_B_PER_W = _B // _NW
_CHUNK = 128
_N_CHUNKS = _B_PER_W // _CHUNK
_NBUF = 4
_LOOKAHEAD = _NBUF // 2

assert _B % _NW == 0 and _B_PER_W % _CHUNK == 0 and _N_CHUNKS > _NBUF


def _gather_sc(table, idx_flat):
    mesh = plsc.VectorSubcoreMesh(core_axis_name="c", subcore_axis_name="s")

    @functools.partial(
        pl.kernel,
        mesh=mesh,
        out_type=jax.ShapeDtypeStruct((_B, 128), jnp.float32),
        scratch_types=[
            pltpu.VMEM((_B_PER_W,), jnp.int32),
        ] + [pltpu.VMEM((_CHUNK, 128), jnp.float32) for _ in range(_NBUF)]
          + [pltpu.SemaphoreType.DMA for _ in range(2 * _NBUF)],
        compiler_params=pltpu.CompilerParams(use_tc_tiling_on_sc=True),
    )
    def k(table_hbm, idx_hbm, out_hbm, idx_all, *bufs_and_sems):
        bufs = bufs_and_sems[:_NBUF]
        gsems = bufs_and_sems[_NBUF:2 * _NBUF]
        wsems = bufs_and_sems[2 * _NBUF:]
        wid = lax.axis_index("s") * _NC + lax.axis_index("c")
        base = wid * _B_PER_W

        pltpu.sync_copy(idx_hbm.at[pl.ds(base, _B_PER_W)], idx_all)

        def start_gather(chunk, b):
            pltpu.make_async_copy(
                table_hbm.at[idx_all.at[pl.ds(chunk * _CHUNK, _CHUNK)]],
                bufs[b], gsems[b]).start()

        def wait_gather(chunk, b):
            pltpu.make_async_copy(
                table_hbm.at[idx_all.at[pl.ds(chunk * _CHUNK, _CHUNK)]],
                bufs[b], gsems[b]).wait()

        def start_write(chunk, b):
            pltpu.make_async_copy(
                bufs[b], out_hbm.at[pl.ds(base + chunk * _CHUNK, _CHUNK)],
                wsems[b]).start()

        def wait_write(chunk, b):
            pltpu.make_async_copy(
                bufs[b], out_hbm.at[pl.ds(base + chunk * _CHUNK, _CHUNK)],
                wsems[b]).wait()

        for j in range(_LOOKAHEAD):
            start_gather(j, j % _NBUF)

        def body(g, carry):
            for bb in range(_NBUF):
                j = g * _NBUF + bb
                wait_gather(j, bb)
                start_write(j, bb)
                t = j + _LOOKAHEAD
                bt = (bb + _LOOKAHEAD) % _NBUF

                @pl.when(t >= _NBUF)
                def _(t=t, bt=bt):
                    wait_write(t - _NBUF, bt)

                @pl.when(t < _N_CHUNKS)
                def _(t=t, bt=bt):
                    start_gather(t, bt)
            return carry

        lax.fori_loop(0, _N_CHUNKS // _NBUF, body, 0)

        for j in range(_N_CHUNKS - _LOOKAHEAD, _N_CHUNKS):
            wait_write(j, j % _NBUF)

    return k(table, idx_flat)


def kernel(input_ids, shared_weight):
    idx_flat = input_ids.reshape(-1).astype(jnp.int32)
    table128 = jnp.pad(shared_weight, ((0, 0), (0, 128 - D_MODEL)))
    out = _gather_sc(table128, idx_flat)[:, :D_MODEL]
    return out.reshape(*input_ids.shape, D_MODEL)
